# Initial kernel scaffold; baseline (speedup 1.0000x reference)
import functools

import jax
import jax.numpy as jnp
from jax import lax
from jax.experimental import pallas as pl
from jax.experimental.pallas import tpu as pltpu
from jax.experimental.pallas import tpu_sc as plsc

N_NODES = 10000
N_EDGES = 320000
D = 128
NUM_RELS = 8

NC = 2
NS = 16
NW = NC * NS
EPW = N_EDGES // NW
CH = 80
STEPS = EPW // CH
ZR = N_NODES // NS


def _node_transform_body(h_ref, op_ref, norm_ref, w_ref, t_ref):
    h = h_ref[...]
    op = op_ref[...]
    norm = norm_ref[...]
    acc = jnp.zeros_like(t_ref)
    for r in range(NUM_RELS):
        scale = jnp.where(op == r, norm, 0.0)
        acc += jnp.dot(h * scale, w_ref[r], preferred_element_type=jnp.float32)
    t_ref[...] = acc


def _node_transform(h, op2, norm2, weight):
    return pl.pallas_call(
        _node_transform_body,
        out_shape=jax.ShapeDtypeStruct((N_NODES, D), jnp.float32),
    )(h, op2, norm2, weight)


_sc_mesh = plsc.VectorSubcoreMesh(
    core_axis_name="c", subcore_axis_name="s", num_cores=NC, num_subcores=NS
)


@functools.partial(
    pl.kernel,
    out_type=jax.ShapeDtypeStruct((NC, N_NODES, D), jnp.float32),
    mesh=_sc_mesh,
    scratch_types=[
        pltpu.VMEM((STEPS, CH), jnp.int32),
        pltpu.VMEM((STEPS, CH), jnp.int32),
        pltpu.VMEM((CH, D), jnp.float32),
        pltpu.VMEM_SHARED((N_NODES, D), jnp.float32),
        pltpu.SemaphoreType.DMA,
    ],
)
def _sc_scatter(t_hbm, src_hbm, dst_hbm, zeros_hbm, out_hbm,
                src_v, dst_v, rows_v, acc, sem):
    cid = lax.axis_index("c")
    sid = lax.axis_index("s")
    wid = sid * NC + cid

    pltpu.sync_copy(zeros_hbm, acc.at[pl.ds(sid * ZR, ZR)])
    pltpu.sync_copy(src_hbm.at[wid], src_v)
    pltpu.sync_copy(dst_hbm.at[wid], dst_v)
    plsc.subcore_barrier()

    @pl.loop(0, STEPS)
    def _step(s):
        pltpu.async_copy(t_hbm.at[src_v.at[s]], rows_v, sem).wait()
        pltpu.sync_copy(rows_v, acc.at[dst_v.at[s]], add=True)

    plsc.subcore_barrier()
    pltpu.sync_copy(acc.at[pl.ds(sid * ZR, ZR)],
                    out_hbm.at[cid, pl.ds(sid * ZR, ZR)])


def _merge_body(p_ref, o_ref):
    o_ref[...] = p_ref[0] + p_ref[1]


def _merge(partials):
    return pl.pallas_call(
        _merge_body,
        out_shape=jax.ShapeDtypeStruct((N_NODES, D), jnp.float32),
    )(partials)


def kernel(h, edge_index, op_class_id, norm, weight):
    src = edge_index[0].astype(jnp.int32).reshape(NW, STEPS, CH)
    dst = edge_index[1].astype(jnp.int32).reshape(NW, STEPS, CH)
    op2 = op_class_id.astype(jnp.int32).reshape(N_NODES, 1)
    norm2 = norm.astype(jnp.float32).reshape(N_NODES, 1)
    t = _node_transform(h, op2, norm2, weight)
    zeros = jnp.zeros((ZR, D), jnp.float32)
    partials = _sc_scatter(t, src, dst, zeros)
    return _merge(partials)


# speedup vs baseline: 7.5459x; 7.5459x over previous
import functools

import jax
import jax.numpy as jnp
from jax import lax
from jax.experimental import pallas as pl
from jax.experimental.pallas import tpu as pltpu
from jax.experimental.pallas import tpu_sc as plsc

N_NODES = 10000
N_EDGES = 320000
D = 128
NUM_RELS = 8

NC = 2
NS = 16
NW = NC * NS
EPW = N_EDGES // NW
CH = 80
STEPS = EPW // CH
NPAD = 10240
ZR = NPAD // NS


def _node_transform_body(h_ref, op_ref, norm_ref, w_ref, t_ref):
    h = h_ref[...]
    op = op_ref[...]
    norm = norm_ref[...]
    acc = jnp.zeros_like(t_ref)
    for r in range(NUM_RELS):
        scale = jnp.where(op == r, norm, 0.0)
        acc += jnp.dot(h * scale, w_ref[r], preferred_element_type=jnp.float32)
    t_ref[...] = acc


def _node_transform(h, op2, norm2, weight):
    return pl.pallas_call(
        _node_transform_body,
        out_shape=jax.ShapeDtypeStruct((N_NODES, D), jnp.float32),
    )(h, op2, norm2, weight)


_sc_mesh = plsc.VectorSubcoreMesh(
    core_axis_name="c", subcore_axis_name="s", num_cores=NC, num_subcores=NS
)


@functools.partial(
    pl.kernel,
    out_type=jax.ShapeDtypeStruct((NC, NPAD, D), jnp.float32),
    mesh=_sc_mesh,
    scratch_types=[
        pltpu.VMEM((STEPS, CH), jnp.int32),
        pltpu.VMEM((STEPS, CH), jnp.int32),
        pltpu.VMEM((CH, D), jnp.float32),
        pltpu.VMEM_SHARED((NPAD, D), jnp.float32),
        pltpu.SemaphoreType.DMA,
    ],
)
def _sc_scatter(t_hbm, src_hbm, dst_hbm, zeros_hbm, out_hbm,
                src_v, dst_v, rows_v, acc, sem):
    cid = lax.axis_index("c")
    sid = lax.axis_index("s")
    wid = sid * NC + cid

    pltpu.sync_copy(zeros_hbm, acc.at[pl.ds(sid * ZR, ZR)])
    pltpu.sync_copy(src_hbm.at[wid], src_v)
    pltpu.sync_copy(dst_hbm.at[wid], dst_v)
    plsc.subcore_barrier()

    @pl.loop(0, STEPS)
    def _step(s):
        pltpu.async_copy(t_hbm.at[src_v.at[s]], rows_v, sem).wait()
        pltpu.sync_copy(rows_v, acc.at[dst_v.at[s]], add=True)

    plsc.subcore_barrier()
    pltpu.sync_copy(acc.at[pl.ds(sid * ZR, ZR)],
                    out_hbm.at[cid, pl.ds(sid * ZR, ZR)])


def _merge_body(p_ref, o_ref):
    o_ref[...] = p_ref[0, :N_NODES, :] + p_ref[1, :N_NODES, :]


def _merge(partials):
    return pl.pallas_call(
        _merge_body,
        out_shape=jax.ShapeDtypeStruct((N_NODES, D), jnp.float32),
    )(partials)


def kernel(h, edge_index, op_class_id, norm, weight):
    src = edge_index[0].astype(jnp.int32).reshape(NW, STEPS, CH)
    dst = edge_index[1].astype(jnp.int32).reshape(NW, STEPS, CH)
    op2 = op_class_id.astype(jnp.int32).reshape(N_NODES, 1)
    norm2 = norm.astype(jnp.float32).reshape(N_NODES, 1)
    t = _node_transform(h, op2, norm2, weight)
    zeros = jnp.zeros((ZR, D), jnp.float32)
    partials = _sc_scatter(t, src, dst, zeros)
    return _merge(partials)
